# baseline (speedup 1.0000x reference)
import functools

import jax
import jax.numpy as jnp
from jax import lax
from jax.experimental import pallas as pl
from jax.experimental.pallas import tpu as pltpu
from jax.experimental.pallas import tpu_sc as plsc

_N_FIELDS = 26
_VOCAB = 1000000
_BATCH = 16384
_NC = 2
_NS = 16
_L = 16
_NW = _NC * _NS
_BPW = _BATCH // _NW
_E = _BPW * _N_FIELDS

_mesh = plsc.VectorSubcoreMesh(
    core_axis_name="c", subcore_axis_name="s",
    num_cores=_NC, num_subcores=_NS)


@functools.partial(
    pl.kernel,
    out_type=jax.ShapeDtypeStruct((_BATCH,), jnp.float32),
    mesh=_mesh,
    scratch_types=[
        pltpu.VMEM((_E,), jnp.int32),
        pltpu.VMEM((_E,), jnp.float32),
        pltpu.VMEM((_N_FIELDS * _L,), jnp.float32),
        pltpu.VMEM((_L,), jnp.float32),
        pltpu.VMEM((_BPW,), jnp.float32),
        pltpu.SemaphoreType.DMA,
    ],
)
def _emb_lr(idxt_hbm, tab_hbm, wbc_hbm, bvec_hbm, out_hbm,
            idx_v, g_v, wbc_v, b_v, out_v, sem):
    wid = lax.axis_index("s") * _NC + lax.axis_index("c")
    base = wid * _BPW
    for f in range(_N_FIELDS):
        pltpu.sync_copy(idxt_hbm.at[pl.ds(f * _BATCH + base, _BPW)],
                        idx_v.at[pl.ds(f * _BPW, _BPW)])
    pltpu.sync_copy(wbc_hbm, wbc_v)
    pltpu.sync_copy(bvec_hbm, b_v)

    def build(i, carry):
        f = i // (_BPW // _L)
        idx_v[pl.ds(i * _L, _L)] = (
            idx_v[pl.ds(i * _L, _L)] + f * _VOCAB)
        return carry
    lax.fori_loop(0, _E // _L, build, 0, unroll=8)

    pltpu.async_copy(tab_hbm.at[idx_v], g_v, sem).wait()

    def accum(c, carry):
        acc = b_v[...]
        for f in range(_N_FIELDS):
            acc = acc + (g_v[pl.ds(f * _BPW + c * _L, _L)]
                         * wbc_v[pl.ds(f * _L, _L)])
        out_v[pl.ds(c * _L, _L)] = 1.0 / (1.0 + jnp.exp(-acc))
        return carry
    lax.fori_loop(0, _BPW // _L, accum, 0)

    pltpu.sync_copy(out_v, out_hbm.at[pl.ds(base, _BPW)])


def kernel(idx, tables, W, b):
    idxt = idx.T.reshape(_N_FIELDS * _BATCH)
    tab_flat = tables.reshape(_N_FIELDS * _VOCAB)
    wbc = jnp.repeat(W.reshape(_N_FIELDS, 1), _L, axis=1).reshape(-1)
    bvec = jnp.broadcast_to(b.reshape(1), (_L,))
    return _emb_lr(idxt, tab_flat, wbc, bvec)


# speedup vs baseline: 1.0046x; 1.0046x over previous
import functools

import jax
import jax.numpy as jnp
from jax import lax
from jax.experimental import pallas as pl
from jax.experimental.pallas import tpu as pltpu
from jax.experimental.pallas import tpu_sc as plsc

_N_FIELDS = 26
_VOCAB = 1000000
_BATCH = 16384
_NC = 2
_NS = 16
_L = 16
_NW = _NC * _NS
_BPW = _BATCH // _NW
_E = _BPW * _N_FIELDS

_mesh = plsc.VectorSubcoreMesh(
    core_axis_name="c", subcore_axis_name="s",
    num_cores=_NC, num_subcores=_NS)


@functools.partial(
    pl.kernel,
    out_type=jax.ShapeDtypeStruct((_BATCH,), jnp.float32),
    mesh=_mesh,
    scratch_types=[
        pltpu.VMEM((_E,), jnp.int32),
        pltpu.VMEM((_E,), jnp.float32),
        pltpu.VMEM((_N_FIELDS * _L,), jnp.float32),
        pltpu.VMEM((_L,), jnp.float32),
        pltpu.VMEM((_BPW,), jnp.float32),
        pltpu.SemaphoreType.DMA,
    ],
)
def _emb_lr(idxt_hbm, tab_hbm, wbc_hbm, bvec_hbm, out_hbm,
            idx_v, g_v, wbc_v, b_v, out_v, sem):
    wid = lax.axis_index("s") * _NC + lax.axis_index("c")
    base = wid * _BPW
    stage = [
        pltpu.async_copy(idxt_hbm.at[pl.ds(f * _BATCH + base, _BPW)],
                         idx_v.at[pl.ds(f * _BPW, _BPW)], sem)
        for f in range(_N_FIELDS)
    ]
    pltpu.sync_copy(wbc_hbm, wbc_v)
    pltpu.sync_copy(bvec_hbm, b_v)
    for d in stage:
        d.wait()

    def build(i, carry):
        f = i // (_BPW // _L)
        idx_v[pl.ds(i * _L, _L)] = (
            idx_v[pl.ds(i * _L, _L)] + f * _VOCAB)
        return carry
    lax.fori_loop(0, _E // _L, build, 0, unroll=8)

    nstream = 8
    chunk = _E // nstream
    gath = [
        pltpu.async_copy(tab_hbm.at[idx_v.at[pl.ds(g * chunk, chunk)]],
                         g_v.at[pl.ds(g * chunk, chunk)], sem)
        for g in range(nstream)
    ]
    for d in gath:
        d.wait()

    def accum(c, carry):
        acc = b_v[...]
        for f in range(_N_FIELDS):
            acc = acc + (g_v[pl.ds(f * _BPW + c * _L, _L)]
                         * wbc_v[pl.ds(f * _L, _L)])
        out_v[pl.ds(c * _L, _L)] = 1.0 / (1.0 + jnp.exp(-acc))
        return carry
    lax.fori_loop(0, _BPW // _L, accum, 0)

    pltpu.sync_copy(out_v, out_hbm.at[pl.ds(base, _BPW)])


def kernel(idx, tables, W, b):
    idxt = idx.T.reshape(_N_FIELDS * _BATCH)
    tab_flat = tables.reshape(_N_FIELDS * _VOCAB)
    wbc = jnp.repeat(W.reshape(_N_FIELDS, 1), _L, axis=1).reshape(-1)
    bvec = jnp.broadcast_to(b.reshape(1), (_L,))
    return _emb_lr(idxt, tab_flat, wbc, bvec)
